# baseline (speedup 1.0000x reference)
import functools

import jax
import jax.numpy as jnp
from jax import lax
from jax.experimental import pallas as pl
from jax.experimental.pallas import tpu as pltpu
from jax.experimental.pallas import tpu_sc as plsc

_NUM_CORES = 2
_NUM_SUBCORES = 16
_NW = _NUM_CORES * _NUM_SUBCORES
_LANES = 16
_CHUNK = 512


@functools.lru_cache(maxsize=None)
def _build(B, D):
    assert B % (_NW * _CHUNK) == 0, (B, D)
    bpw = B // _NW
    nchunk = bpw // _CHUNK
    mesh = plsc.VectorSubcoreMesh(core_axis_name="c", subcore_axis_name="s")

    @functools.partial(
        pl.kernel,
        out_type=jax.ShapeDtypeStruct((B, D), jnp.float32),
        mesh=mesh,
        scratch_types=[
            pltpu.VMEM((_CHUNK,), jnp.int32),
            pltpu.VMEM((_CHUNK, D), jnp.float32),
            pltpu.SemaphoreType.DMA,
        ],
        compiler_params=pltpu.CompilerParams(use_tc_tiling_on_sc=False),
    )
    def embed(ids_hbm, table_hbm, out_hbm, idx_v, rows_v, sem):
        wid = lax.axis_index("s") * _NUM_CORES + lax.axis_index("c")
        base = wid * bpw

        def chunk_body(g, carry):
            off = base + g * _CHUNK
            pltpu.sync_copy(ids_hbm.at[pl.ds(off, _CHUNK)], idx_v)

            def clamp_body(i, c):
                sl = pl.ds(i * _LANES, _LANES)
                idx_v[sl] = jnp.maximum(idx_v[sl], 0)
                return c

            lax.fori_loop(0, _CHUNK // _LANES, clamp_body, 0)
            pltpu.async_copy(table_hbm.at[idx_v], rows_v, sem).wait()
            pltpu.sync_copy(rows_v, out_hbm.at[pl.ds(off, _CHUNK)])
            return carry

        lax.fori_loop(0, nchunk, chunk_body, 0)

    return embed


def kernel(token_ids, table):
    bsz, seq = token_ids.shape
    vocab, dim = table.shape
    ids = token_ids.reshape(-1)
    out = _build(bsz * seq, dim)(ids, table)
    return out.reshape(bsz, seq, dim)


# speedup vs baseline: 1.0474x; 1.0474x over previous
import functools

import jax
import jax.numpy as jnp
from jax import lax
from jax.experimental import pallas as pl
from jax.experimental.pallas import tpu as pltpu
from jax.experimental.pallas import tpu_sc as plsc

_NUM_CORES = 2
_NUM_SUBCORES = 16
_NW = _NUM_CORES * _NUM_SUBCORES
_LANES = 16
_CHUNK = 400
_NBUF = 4


@functools.lru_cache(maxsize=None)
def _build(B, D):
    assert B % (_NW * _CHUNK) == 0, (B, D)
    bpw = B // _NW
    nchunk = bpw // _CHUNK
    assert nchunk % _NBUF == 0 and nchunk >= 2 * _NBUF
    mesh = plsc.VectorSubcoreMesh(core_axis_name="c", subcore_axis_name="s")

    @functools.partial(
        pl.kernel,
        out_type=jax.ShapeDtypeStruct((B, D), jnp.float32),
        mesh=mesh,
        scratch_types=[
            pltpu.VMEM((bpw,), jnp.int32),
            [pltpu.VMEM((_CHUNK, D), jnp.float32) for _ in range(_NBUF)],
            [pltpu.SemaphoreType.DMA for _ in range(_NBUF)],
            [pltpu.SemaphoreType.DMA for _ in range(_NBUF)],
        ],
        compiler_params=pltpu.CompilerParams(use_tc_tiling_on_sc=False),
    )
    def embed(ids_hbm, table_hbm, out_hbm, ids_v, rows, gsem, ssem):
        wid = lax.axis_index("s") * _NUM_CORES + lax.axis_index("c")
        base = wid * bpw
        pltpu.sync_copy(ids_hbm.at[pl.ds(base, bpw)], ids_v)

        def clamp_chunk(g):
            @pl.loop(0, _CHUNK // _LANES)
            def _(i):
                sl = pl.ds(g * _CHUNK + i * _LANES, _LANES)
                ids_v[sl] = jnp.maximum(ids_v[sl], 0)

        def start_gather(g, b):
            pltpu.async_copy(
                table_hbm.at[ids_v.at[pl.ds(g * _CHUNK, _CHUNK)]], rows[b], gsem[b]
            )

        def wait_gather(b):
            pltpu.make_async_copy(
                table_hbm.at[ids_v.at[pl.ds(0, _CHUNK)]], rows[b], gsem[b]
            ).wait()

        def start_store(g, b):
            pltpu.async_copy(
                rows[b], out_hbm.at[pl.ds(base + g * _CHUNK, _CHUNK)], ssem[b]
            )

        def wait_store(b):
            pltpu.make_async_copy(
                rows[b], out_hbm.at[pl.ds(base, _CHUNK)], ssem[b]
            ).wait()

        for b in range(_NBUF):
            clamp_chunk(b)
            start_gather(b, b)

        @pl.loop(0, nchunk - _NBUF, step=_NBUF)
        def _(g0):
            for b in range(_NBUF):
                g = g0 + b
                wait_gather(b)
                start_store(g, b)
                clamp_chunk(g + _NBUF)
                wait_store(b)
                start_gather(g + _NBUF, b)

        for b in range(_NBUF):
            g = nchunk - _NBUF + b
            wait_gather(b)
            start_store(g, b)
        for b in range(_NBUF):
            wait_store(b)

    return embed


def kernel(token_ids, table):
    bsz, seq = token_ids.shape
    vocab, dim = table.shape
    ids = token_ids.reshape(-1)
    out = _build(bsz * seq, dim)(ids, table)
    return out.reshape(bsz, seq, dim)


# speedup vs baseline: 1.0518x; 1.0042x over previous
import functools

import jax
import jax.numpy as jnp
from jax import lax
from jax.experimental import pallas as pl
from jax.experimental.pallas import tpu as pltpu
from jax.experimental.pallas import tpu_sc as plsc

_NUM_CORES = 2
_NUM_SUBCORES = 16
_NW = _NUM_CORES * _NUM_SUBCORES
_LANES = 16
_NBUF = 4


@functools.lru_cache(maxsize=None)
def _build(BSZ, SEQ, D):
    assert BSZ % _NW == 0, (BSZ, SEQ, D)
    rpw = BSZ // _NW
    assert rpw % _NBUF == 0 and rpw >= 2 * _NBUF
    clamp_offs = list(range(0, SEQ - _LANES + 1, _LANES))
    if SEQ % _LANES:
        clamp_offs.append(SEQ - _LANES)
    mesh = plsc.VectorSubcoreMesh(core_axis_name="c", subcore_axis_name="s")

    @functools.partial(
        pl.kernel,
        out_type=jax.ShapeDtypeStruct((BSZ, SEQ, D), jnp.float32),
        mesh=mesh,
        scratch_types=[
            pltpu.VMEM((rpw, SEQ), jnp.int32),
            [pltpu.VMEM((SEQ, D), jnp.float32) for _ in range(_NBUF)],
            [pltpu.SemaphoreType.DMA for _ in range(_NBUF)],
            [pltpu.SemaphoreType.DMA for _ in range(_NBUF)],
        ],
        compiler_params=pltpu.CompilerParams(use_tc_tiling_on_sc=False),
    )
    def embed(ids_hbm, table_hbm, out_hbm, ids_v, rows, gsem, ssem):
        wid = lax.axis_index("s") * _NUM_CORES + lax.axis_index("c")
        base = wid * rpw
        pltpu.sync_copy(ids_hbm.at[pl.ds(base, rpw)], ids_v)

        def clamp_row(r):
            for off in clamp_offs:
                sl = pl.ds(off, _LANES)
                ids_v[r, sl] = jnp.maximum(ids_v[r, sl], 0)

        def start_gather(r, b):
            pltpu.async_copy(table_hbm.at[ids_v.at[r]], rows[b], gsem[b])

        def wait_gather(b):
            pltpu.make_async_copy(
                table_hbm.at[ids_v.at[0]], rows[b], gsem[b]
            ).wait()

        def start_store(r, b):
            pltpu.async_copy(rows[b], out_hbm.at[base + r], ssem[b])

        def wait_store(b):
            pltpu.make_async_copy(rows[b], out_hbm.at[base], ssem[b]).wait()

        for b in range(_NBUF):
            clamp_row(b)
            start_gather(b, b)

        @pl.loop(0, rpw - _NBUF, step=_NBUF)
        def _(r0):
            for b in range(_NBUF):
                r = r0 + b
                wait_gather(b)
                start_store(r, b)
                clamp_row(r + _NBUF)
                wait_store(b)
                start_gather(r + _NBUF, b)

        for b in range(_NBUF):
            r = rpw - _NBUF + b
            wait_gather(b)
            start_store(r, b)
        for b in range(_NBUF):
            wait_store(b)

    return embed


def kernel(token_ids, table):
    bsz, seq = token_ids.shape
    vocab, dim = table.shape
    return _build(bsz, seq, dim)(token_ids, table)
